# baseline (speedup 1.0000x reference)
import functools

import jax
import jax.numpy as jnp
from jax import lax
from jax.experimental import pallas as pl
from jax.experimental.pallas import tpu as pltpu
from jax.experimental.pallas import tpu_sc as plsc

NFEAT = 26
VOCAB = 100000
EMBED = 32
BATCH = 16384

NC = 2
NS = 16
NW = NC * NS
NCOL = NFEAT * EMBED
COLS_PW = NCOL // NW
OCH = 4096
NCHUNK = BATCH // OCH
GRP = OCH // 16
UNROLL = 16

_mesh = plsc.VectorSubcoreMesh(core_axis_name="c", subcore_axis_name="s")


@functools.partial(
    pl.kernel,
    mesh=_mesh,
    out_type=jax.ShapeDtypeStruct((NCOL, BATCH), jnp.float32),
    scratch_types=[
        pltpu.VMEM((VOCAB,), jnp.float32),
        pltpu.VMEM((BATCH,), jnp.int32),
        pltpu.VMEM((2, OCH), jnp.float32),
        pltpu.SemaphoreType.DMA,
    ],
    compiler_params=pltpu.CompilerParams(needs_layout_passes=False),
)
def _emb_lookup(x_hbm, tab_hbm, out_hbm, vocab_v, idx_v, outbuf, sem_out):
    wid = lax.axis_index("s") * NC + lax.axis_index("c")
    c0 = wid * COLS_PW

    def per_column(ci, _):
        c = c0 + ci
        f = c // EMBED
        e = c % EMBED

        @pl.when(jnp.logical_or(ci == 0, e == 0))
        def _():
            pltpu.sync_copy(x_hbm.at[f], idx_v)

        pltpu.sync_copy(tab_hbm.at[f, e], vocab_v)

        def per_chunk(k, _):
            g = ci * NCHUNK + k

            @pl.when(g >= 2)
            def _():
                pltpu.make_async_copy(
                    outbuf.at[0], out_hbm.at[c, pl.ds(0, OCH)], sem_out
                ).wait()

            buf = k % 2

            @plsc.parallel_loop(0, GRP, 1, unroll=UNROLL)
            def _(j):
                base = k * OCH + j * 16
                iv = idx_v[pl.ds(base, 16)]
                outbuf[buf, pl.ds(j * 16, 16)] = plsc.load_gather(
                    vocab_v, [iv]
                )
            pltpu.async_copy(
                outbuf.at[buf], out_hbm.at[c, pl.ds(k * OCH, OCH)], sem_out
            )
            return 0

        lax.fori_loop(0, NCHUNK, per_chunk, 0)
        return 0

    lax.fori_loop(0, COLS_PW, per_column, 0)

    for _ in range(2):
        pltpu.make_async_copy(
            outbuf.at[0], out_hbm.at[c0, pl.ds(0, OCH)], sem_out
        ).wait()


def kernel(x, tables):
    t2 = jnp.transpose(tables, (0, 2, 1))
    out = _emb_lookup(x.astype(jnp.int32), t2)
    return out.T


# speedup vs baseline: 1.0608x; 1.0608x over previous
import functools

import jax
import jax.numpy as jnp
from jax import lax
from jax.experimental import pallas as pl
from jax.experimental.pallas import tpu as pltpu
from jax.experimental.pallas import tpu_sc as plsc

NFEAT = 26
VOCAB = 100000
EMBED = 32
BATCH = 16384

NC = 2
NS = 16
NW = NC * NS
NCOL = NFEAT * EMBED
COLS_PW = NCOL // NW
OCH = 2048
NCHUNK = BATCH // OCH
GRP = OCH // 16
UNROLL = 16

_mesh = plsc.VectorSubcoreMesh(core_axis_name="c", subcore_axis_name="s")


@functools.partial(
    pl.kernel,
    mesh=_mesh,
    out_type=jax.ShapeDtypeStruct((NCOL, BATCH), jnp.float32),
    scratch_types=[
        pltpu.VMEM((VOCAB,), jnp.float32),
        pltpu.VMEM((BATCH,), jnp.int32),
        pltpu.VMEM((2, OCH), jnp.float32),
        pltpu.SemaphoreType.DMA,
    ],
    compiler_params=pltpu.CompilerParams(needs_layout_passes=False),
)
def _emb_lookup(x_hbm, tab_hbm, out_hbm, vocab_v, idx_v, outbuf, sem_out):
    wid = lax.axis_index("s") * NC + lax.axis_index("c")
    c0 = wid * COLS_PW

    def per_column(ci, _):
        c = c0 + ci
        f = c // EMBED
        e = c % EMBED

        @pl.when(jnp.logical_or(ci == 0, e == 0))
        def _():
            pltpu.sync_copy(x_hbm.at[f], idx_v)

        pltpu.sync_copy(tab_hbm.at[f, e], vocab_v)

        def per_chunk(k, _):
            g = ci * NCHUNK + k

            @pl.when(g >= 2)
            def _():
                pltpu.make_async_copy(
                    outbuf.at[0], out_hbm.at[c, pl.ds(0, OCH)], sem_out
                ).wait()

            buf = k % 2

            @plsc.parallel_loop(0, GRP, 1, unroll=UNROLL)
            def _(j):
                base = k * OCH + j * 16
                iv = idx_v[pl.ds(base, 16)]
                outbuf[buf, pl.ds(j * 16, 16)] = plsc.load_gather(
                    vocab_v, [iv]
                )
            pltpu.async_copy(
                outbuf.at[buf], out_hbm.at[c, pl.ds(k * OCH, OCH)], sem_out
            )
            return 0

        lax.fori_loop(0, NCHUNK, per_chunk, 0)
        return 0

    lax.fori_loop(0, COLS_PW, per_column, 0)

    for _ in range(2):
        pltpu.make_async_copy(
            outbuf.at[0], out_hbm.at[c0, pl.ds(0, OCH)], sem_out
        ).wait()


def kernel(x, tables):
    t2 = jnp.transpose(tables, (0, 2, 1))
    out = _emb_lookup(x.astype(jnp.int32), t2)
    return out.T
